# baseline (speedup 1.0000x reference)
import functools

import jax
import jax.numpy as jnp
from jax import lax
from jax.experimental import pallas as pl
from jax.experimental.pallas import tpu as pltpu
from jax.experimental.pallas import tpu_sc as plsc

N = 10000
E = 320000
D = 128
NPAD = 10240
NC = 2
NS = 16
CH = 40
DCH = 80
NCH_W = E // (NC * NS * CH)
NCH_T = E // (NS * DCH)
TILE_ROWS = NPAD // NS
HALF = NPAD // NC
EXP_ROWS = NPAD // (NC * NS)
DEG_GRP = 10

_mesh = plsc.VectorSubcoreMesh(core_axis_name="c", subcore_axis_name="s")


def _fill_zero_rows(ref, nrows):
    z = jnp.zeros((16,), jnp.float32)

    def body(r, carry):
        for c in range(D // 16):
            ref[r, pl.ds(c * 16, 16)] = z
        return carry

    lax.fori_loop(0, nrows, body, 0)


@functools.partial(
    pl.kernel,
    out_type=jax.ShapeDtypeStruct((NPAD, D), jnp.float32),
    mesh=_mesh,
    scratch_types=[
        pltpu.VMEM((NCH_T, DCH), jnp.int32),
        pltpu.VMEM((DCH,), jnp.float32),
        pltpu.VMEM((TILE_ROWS,), jnp.float32),
        pltpu.VMEM((EXP_ROWS,), jnp.float32),
        pltpu.VMEM((EXP_ROWS, D), jnp.float32),
        pltpu.VMEM_SHARED((NPAD,), jnp.float32),
        pltpu.SemaphoreType.DMA,
    ],
)
def _deg_kernel(dst_hbm, out_hbm, di_v, ones_v, zb_v, degs_v, exp_v, deg_sh, sem):
    cid = lax.axis_index("c")
    sid = lax.axis_index("s")
    pltpu.sync_copy(dst_hbm.at[sid], di_v)
    one = jnp.full((16,), 1.0, jnp.float32)
    z = jnp.zeros((16,), jnp.float32)
    for k in range(DCH // 16):
        ones_v[pl.ds(k * 16, 16)] = one
    for k in range(TILE_ROWS // 16):
        zb_v[pl.ds(k * 16, 16)] = z
    pltpu.sync_copy(zb_v, deg_sh.at[pl.ds(sid * TILE_ROWS, TILE_ROWS)])
    plsc.subcore_barrier()

    def grp(g, carry):
        for k in range(DEG_GRP):
            pltpu.async_copy(
                ones_v, deg_sh.at[di_v.at[g * DEG_GRP + k]], sem, add=True)
        for k in range(DEG_GRP):
            pltpu.make_async_copy(
                ones_v, deg_sh.at[di_v.at[g * DEG_GRP + k]], sem).wait()
        return carry

    lax.fori_loop(0, NCH_T // DEG_GRP, grp, 0)
    plsc.subcore_barrier()
    base = cid * HALF + sid * EXP_ROWS
    pltpu.sync_copy(deg_sh.at[pl.ds(base, EXP_ROWS)], degs_v)

    def expand(q, carry):
        vec = degs_v[pl.ds(q * 16, 16)] + 1.0
        for k in range(16):
            bc = jnp.full((16,), vec[k], jnp.float32)
            for c in range(D // 16):
                exp_v[q * 16 + k, pl.ds(c * 16, 16)] = bc
        return carry

    lax.fori_loop(0, EXP_ROWS // 16, expand, 0)
    pltpu.sync_copy(exp_v, out_hbm.at[pl.ds(base, EXP_ROWS)])


NSTG = 10
NCHS = NCH_W // NSTG
RING = 5


@functools.partial(
    pl.kernel,
    out_type=jax.ShapeDtypeStruct((NC, NPAD, D), jnp.float32),
    mesh=_mesh,
    scratch_types=[
        pltpu.VMEM((NCHS, CH), jnp.int32),
        pltpu.VMEM((NCHS, CH), jnp.int32),
        pltpu.VMEM((RING, CH, D), jnp.float32),
        pltpu.VMEM_SHARED((NPAD, D), jnp.float32),
        pltpu.SemaphoreType.DMA((RING,)),
    ],
)
def _segsum_kernel(src_hbm, dst_hbm, y_hbm, out_hbm,
                   si_v, di_v, ring_v, acc_sh, semg):
    cid = lax.axis_index("c")
    sid = lax.axis_index("s")
    wid = sid * NC + cid
    _fill_zero_rows(ring_v.at[0], CH)
    for k in range(TILE_ROWS // CH):
        pltpu.sync_copy(ring_v.at[0],
                        acc_sh.at[pl.ds(sid * TILE_ROWS + k * CH, CH)])
    plsc.subcore_barrier()

    for stg in range(NSTG):
        pltpu.sync_copy(src_hbm.at[wid, stg], si_v)
        pltpu.sync_copy(dst_hbm.at[wid, stg], di_v)
        for k in range(RING - 1):
            pltpu.async_copy(y_hbm.at[si_v.at[k]], ring_v.at[k], semg.at[k])

        def itr(g, carry):
            for k in range(RING):
                j = RING * g + k
                bn = (k + RING - 1) % RING

                @pl.when(j + RING - 1 < NCHS)
                def _():
                    pltpu.async_copy(y_hbm.at[si_v.at[j + RING - 1]],
                                     ring_v.at[bn], semg.at[bn])

                pltpu.make_async_copy(y_hbm.at[si_v.at[j]],
                                      ring_v.at[k], semg.at[k]).wait()
                pltpu.sync_copy(ring_v.at[k], acc_sh.at[di_v.at[j]], add=True)
            return carry

        lax.fori_loop(0, NCHS // RING, itr, 0)

    plsc.subcore_barrier()
    pltpu.sync_copy(acc_sh.at[pl.ds(sid * TILE_ROWS, TILE_ROWS)],
                    out_hbm.at[cid, pl.ds(sid * TILE_ROWS, TILE_ROWS)])


RB = 1280
TC_GRID = NPAD // RB


def _tc_scale_mm(x_ref, w_ref, deg_ref, y_ref):
    dinv = lax.rsqrt(deg_ref[...])
    y_ref[...] = jnp.dot(x_ref[...], w_ref[...],
                         preferred_element_type=jnp.float32) * dinv


_scale_mm = pl.pallas_call(
    _tc_scale_mm,
    grid=(TC_GRID,),
    in_specs=[pl.BlockSpec((RB, D), lambda i: (i, 0)),
              pl.BlockSpec((D, D), lambda i: (0, 0)),
              pl.BlockSpec((RB, D), lambda i: (i, 0))],
    out_specs=pl.BlockSpec((RB, D), lambda i: (i, 0)),
    out_shape=jax.ShapeDtypeStruct((NPAD, D), jnp.float32),
)


def _tc_mid(acc_ref, y_ref, deg_ref, b_ref, w_ref, out_ref):
    dinv = lax.rsqrt(deg_ref[...])
    h = jnp.maximum(
        (acc_ref[0] + acc_ref[1] + y_ref[...]) * dinv + b_ref[...], 0.0)
    out_ref[...] = jnp.dot(h, w_ref[...],
                           preferred_element_type=jnp.float32) * dinv


_mid = pl.pallas_call(
    _tc_mid,
    grid=(TC_GRID,),
    in_specs=[pl.BlockSpec((NC, RB, D), lambda i: (0, i, 0)),
              pl.BlockSpec((RB, D), lambda i: (i, 0)),
              pl.BlockSpec((RB, D), lambda i: (i, 0)),
              pl.BlockSpec((1, D), lambda i: (0, 0)),
              pl.BlockSpec((D, D), lambda i: (0, 0))],
    out_specs=pl.BlockSpec((RB, D), lambda i: (i, 0)),
    out_shape=jax.ShapeDtypeStruct((NPAD, D), jnp.float32),
)


def _tc_final(acc_ref, y_ref, deg_ref, b_ref, out_ref):
    dinv = lax.rsqrt(deg_ref[...])
    out_ref[...] = jnp.maximum(
        (acc_ref[0] + acc_ref[1] + y_ref[...]) * dinv + b_ref[...], 0.0)


_final = pl.pallas_call(
    _tc_final,
    grid=(TC_GRID,),
    in_specs=[pl.BlockSpec((NC, RB, D), lambda i: (0, i, 0)),
              pl.BlockSpec((RB, D), lambda i: (i, 0)),
              pl.BlockSpec((RB, D), lambda i: (i, 0)),
              pl.BlockSpec((1, D), lambda i: (0, 0))],
    out_specs=pl.BlockSpec((RB, D), lambda i: (i, 0)),
    out_shape=jax.ShapeDtypeStruct((NPAD, D), jnp.float32),
)


def kernel(x, edge_index, W1, b1, W2, b2):
    src4d = edge_index[0].reshape(NC * NS, NSTG, NCHS, CH)
    dst4d = edge_index[1].reshape(NC * NS, NSTG, NCHS, CH)
    dst3d_t = edge_index[1].reshape(NS, NCH_T, DCH)
    xp = jnp.zeros((NPAD, D), jnp.float32).at[:N].set(x)
    deg = _deg_kernel(dst3d_t)
    y1 = _scale_mm(xp, W1, deg)
    acc1 = _segsum_kernel(src4d, dst4d, y1)
    y2 = _mid(acc1, y1, deg, b1.reshape(1, D), W2)
    acc2 = _segsum_kernel(src4d, dst4d, y2)
    out = _final(acc2, y2, deg, b2.reshape(1, D))
    return out[:N]


# speedup vs baseline: 1.0921x; 1.0921x over previous
import functools

import jax
import jax.numpy as jnp
from jax import lax
from jax.experimental import pallas as pl
from jax.experimental.pallas import tpu as pltpu
from jax.experimental.pallas import tpu_sc as plsc

N = 10000
E = 320000
D = 128
NPAD = 10240
NC = 2
NS = 16
CH = 80
DCH = 80
NCH_W = E // (NC * NS * CH)
NCH_T = E // (NS * DCH)
TILE_ROWS = NPAD // NS
HALF = NPAD // NC
EXP_ROWS = NPAD // (NC * NS)
DEG_GRP = 10

_mesh = plsc.VectorSubcoreMesh(core_axis_name="c", subcore_axis_name="s")


def _fill_zero_rows(ref, nrows):
    z = jnp.zeros((16,), jnp.float32)

    def body(r, carry):
        for c in range(D // 16):
            ref[r, pl.ds(c * 16, 16)] = z
        return carry

    lax.fori_loop(0, nrows, body, 0)


@functools.partial(
    pl.kernel,
    out_type=jax.ShapeDtypeStruct((NPAD, D), jnp.float32),
    mesh=_mesh,
    scratch_types=[
        pltpu.VMEM((NCH_T, DCH), jnp.int32),
        pltpu.VMEM((DCH,), jnp.float32),
        pltpu.VMEM((TILE_ROWS,), jnp.float32),
        pltpu.VMEM((EXP_ROWS,), jnp.float32),
        pltpu.VMEM((EXP_ROWS, D), jnp.float32),
        pltpu.VMEM_SHARED((NPAD,), jnp.float32),
        pltpu.SemaphoreType.DMA,
    ],
)
def _deg_kernel(dst_hbm, out_hbm, di_v, ones_v, zb_v, degs_v, exp_v, deg_sh, sem):
    cid = lax.axis_index("c")
    sid = lax.axis_index("s")
    pltpu.sync_copy(dst_hbm.at[sid], di_v)
    one = jnp.full((16,), 1.0, jnp.float32)
    z = jnp.zeros((16,), jnp.float32)
    for k in range(DCH // 16):
        ones_v[pl.ds(k * 16, 16)] = one
    for k in range(TILE_ROWS // 16):
        zb_v[pl.ds(k * 16, 16)] = z
    pltpu.sync_copy(zb_v, deg_sh.at[pl.ds(sid * TILE_ROWS, TILE_ROWS)])
    plsc.subcore_barrier()

    def grp(g, carry):
        for k in range(DEG_GRP):
            pltpu.async_copy(
                ones_v, deg_sh.at[di_v.at[g * DEG_GRP + k]], sem, add=True)
        for k in range(DEG_GRP):
            pltpu.make_async_copy(
                ones_v, deg_sh.at[di_v.at[g * DEG_GRP + k]], sem).wait()
        return carry

    lax.fori_loop(0, NCH_T // DEG_GRP, grp, 0)
    plsc.subcore_barrier()
    base = cid * HALF + sid * EXP_ROWS
    pltpu.sync_copy(deg_sh.at[pl.ds(base, EXP_ROWS)], degs_v)

    def expand(q, carry):
        vec = degs_v[pl.ds(q * 16, 16)] + 1.0
        for k in range(16):
            bc = jnp.full((16,), vec[k], jnp.float32)
            for c in range(D // 16):
                exp_v[q * 16 + k, pl.ds(c * 16, 16)] = bc
        return carry

    lax.fori_loop(0, EXP_ROWS // 16, expand, 0)
    pltpu.sync_copy(exp_v, out_hbm.at[pl.ds(base, EXP_ROWS)])


NSTG = 5
NCHS = NCH_W // NSTG
RING = 4


@functools.partial(
    pl.kernel,
    out_type=jax.ShapeDtypeStruct((NC, NPAD, D), jnp.float32),
    mesh=_mesh,
    scratch_types=[
        pltpu.VMEM((NCHS, CH), jnp.int32),
        pltpu.VMEM((NCHS, CH), jnp.int32),
        pltpu.VMEM((RING, CH, D), jnp.float32),
        pltpu.VMEM_SHARED((NPAD, D), jnp.float32),
        pltpu.SemaphoreType.DMA((RING,)),
    ],
)
def _segsum_kernel(src_hbm, dst_hbm, y_hbm, out_hbm,
                   si_v, di_v, ring_v, acc_sh, semg):
    cid = lax.axis_index("c")
    sid = lax.axis_index("s")
    wid = sid * NC + cid
    _fill_zero_rows(ring_v.at[0], CH)
    for k in range(TILE_ROWS // CH):
        pltpu.sync_copy(ring_v.at[0],
                        acc_sh.at[pl.ds(sid * TILE_ROWS + k * CH, CH)])
    plsc.subcore_barrier()

    for stg in range(NSTG):
        pltpu.sync_copy(src_hbm.at[wid, stg], si_v)
        pltpu.sync_copy(dst_hbm.at[wid, stg], di_v)
        for k in range(RING - 1):
            pltpu.async_copy(y_hbm.at[si_v.at[k]], ring_v.at[k], semg.at[k])

        def itr(g, carry):
            for k in range(RING):
                j = RING * g + k
                bn = (k + RING - 1) % RING

                @pl.when(j + RING - 1 < NCHS)
                def _():
                    pltpu.async_copy(y_hbm.at[si_v.at[j + RING - 1]],
                                     ring_v.at[bn], semg.at[bn])

                pltpu.make_async_copy(y_hbm.at[si_v.at[j]],
                                      ring_v.at[k], semg.at[k]).wait()
                pltpu.sync_copy(ring_v.at[k], acc_sh.at[di_v.at[j]], add=True)
            return carry

        lax.fori_loop(0, NCHS // RING, itr, 0)
        pltpu.make_async_copy(y_hbm.at[si_v.at[NCHS - 1]],
                              ring_v.at[0], semg.at[0]).wait()
        pltpu.sync_copy(ring_v.at[0], acc_sh.at[di_v.at[NCHS - 1]], add=True)

    plsc.subcore_barrier()
    pltpu.sync_copy(acc_sh.at[pl.ds(sid * TILE_ROWS, TILE_ROWS)],
                    out_hbm.at[cid, pl.ds(sid * TILE_ROWS, TILE_ROWS)])


RB = 1280
TC_GRID = NPAD // RB


def _tc_scale_mm(x_ref, w_ref, deg_ref, y_ref):
    dinv = lax.rsqrt(deg_ref[...])
    y_ref[...] = jnp.dot(x_ref[...], w_ref[...],
                         preferred_element_type=jnp.float32) * dinv


_scale_mm = pl.pallas_call(
    _tc_scale_mm,
    grid=(TC_GRID,),
    in_specs=[pl.BlockSpec((RB, D), lambda i: (i, 0)),
              pl.BlockSpec((D, D), lambda i: (0, 0)),
              pl.BlockSpec((RB, D), lambda i: (i, 0))],
    out_specs=pl.BlockSpec((RB, D), lambda i: (i, 0)),
    out_shape=jax.ShapeDtypeStruct((NPAD, D), jnp.float32),
)


def _tc_mid(acc_ref, y_ref, deg_ref, b_ref, w_ref, out_ref):
    dinv = lax.rsqrt(deg_ref[...])
    h = jnp.maximum(
        (acc_ref[0] + acc_ref[1] + y_ref[...]) * dinv + b_ref[...], 0.0)
    out_ref[...] = jnp.dot(h, w_ref[...],
                           preferred_element_type=jnp.float32) * dinv


_mid = pl.pallas_call(
    _tc_mid,
    grid=(TC_GRID,),
    in_specs=[pl.BlockSpec((NC, RB, D), lambda i: (0, i, 0)),
              pl.BlockSpec((RB, D), lambda i: (i, 0)),
              pl.BlockSpec((RB, D), lambda i: (i, 0)),
              pl.BlockSpec((1, D), lambda i: (0, 0)),
              pl.BlockSpec((D, D), lambda i: (0, 0))],
    out_specs=pl.BlockSpec((RB, D), lambda i: (i, 0)),
    out_shape=jax.ShapeDtypeStruct((NPAD, D), jnp.float32),
)


def _tc_final(acc_ref, y_ref, deg_ref, b_ref, out_ref):
    dinv = lax.rsqrt(deg_ref[...])
    out_ref[...] = jnp.maximum(
        (acc_ref[0] + acc_ref[1] + y_ref[...]) * dinv + b_ref[...], 0.0)


_final = pl.pallas_call(
    _tc_final,
    grid=(TC_GRID,),
    in_specs=[pl.BlockSpec((NC, RB, D), lambda i: (0, i, 0)),
              pl.BlockSpec((RB, D), lambda i: (i, 0)),
              pl.BlockSpec((RB, D), lambda i: (i, 0)),
              pl.BlockSpec((1, D), lambda i: (0, 0))],
    out_specs=pl.BlockSpec((RB, D), lambda i: (i, 0)),
    out_shape=jax.ShapeDtypeStruct((NPAD, D), jnp.float32),
)


def kernel(x, edge_index, W1, b1, W2, b2):
    src4d = edge_index[0].reshape(NC * NS, NSTG, NCHS, CH)
    dst4d = edge_index[1].reshape(NC * NS, NSTG, NCHS, CH)
    dst3d_t = edge_index[1].reshape(NS, NCH_T, DCH)
    xp = jnp.zeros((NPAD, D), jnp.float32).at[:N].set(x)
    deg = _deg_kernel(dst3d_t)
    y1 = _scale_mm(xp, W1, deg)
    acc1 = _segsum_kernel(src4d, dst4d, y1)
    y2 = _mid(acc1, y1, deg, b1.reshape(1, D), W2)
    acc2 = _segsum_kernel(src4d, dst4d, y2)
    out = _final(acc2, y2, deg, b2.reshape(1, D))
    return out[:N]


# speedup vs baseline: 1.1091x; 1.0156x over previous
import functools

import jax
import jax.numpy as jnp
from jax import lax
from jax.experimental import pallas as pl
from jax.experimental.pallas import tpu as pltpu
from jax.experimental.pallas import tpu_sc as plsc

N = 10000
E = 320000
D = 128
NPAD = 10240
NC = 2
NS = 16
CH = 80
DCH = 80
NCH_W = E // (NC * NS * CH)
NCH_T = E // (NS * DCH)
TILE_ROWS = NPAD // NS
HALF = NPAD // NC
EXP_ROWS = NPAD // (NC * NS)
DEG_GRP = 10

_mesh = plsc.VectorSubcoreMesh(core_axis_name="c", subcore_axis_name="s")


def _fill_zero_rows(ref, nrows):
    z = jnp.zeros((16,), jnp.float32)

    def body(r, carry):
        for c in range(D // 16):
            ref[r, pl.ds(c * 16, 16)] = z
        return carry

    lax.fori_loop(0, nrows, body, 0)


@functools.partial(
    pl.kernel,
    out_type=jax.ShapeDtypeStruct((NPAD, D), jnp.float32),
    mesh=_mesh,
    scratch_types=[
        pltpu.VMEM((NCH_T, DCH), jnp.int32),
        pltpu.VMEM((DCH,), jnp.float32),
        pltpu.VMEM((TILE_ROWS,), jnp.float32),
        pltpu.VMEM((EXP_ROWS,), jnp.float32),
        pltpu.VMEM((EXP_ROWS, D), jnp.float32),
        pltpu.VMEM_SHARED((NPAD,), jnp.float32),
        pltpu.SemaphoreType.DMA,
    ],
)
def _deg_kernel(dst_hbm, out_hbm, di_v, ones_v, zb_v, degs_v, exp_v, deg_sh, sem):
    cid = lax.axis_index("c")
    sid = lax.axis_index("s")
    pltpu.sync_copy(dst_hbm.at[sid], di_v)
    one = jnp.full((16,), 1.0, jnp.float32)
    z = jnp.zeros((16,), jnp.float32)
    for k in range(DCH // 16):
        ones_v[pl.ds(k * 16, 16)] = one
    for k in range(TILE_ROWS // 16):
        zb_v[pl.ds(k * 16, 16)] = z
    pltpu.sync_copy(zb_v, deg_sh.at[pl.ds(sid * TILE_ROWS, TILE_ROWS)])
    plsc.subcore_barrier()

    def grp(g, carry):
        for k in range(DEG_GRP):
            pltpu.async_copy(
                ones_v, deg_sh.at[di_v.at[g * DEG_GRP + k]], sem, add=True)
        for k in range(DEG_GRP):
            pltpu.make_async_copy(
                ones_v, deg_sh.at[di_v.at[g * DEG_GRP + k]], sem).wait()
        return carry

    lax.fori_loop(0, NCH_T // DEG_GRP, grp, 0)
    plsc.subcore_barrier()
    base = cid * HALF + sid * EXP_ROWS
    pltpu.sync_copy(deg_sh.at[pl.ds(base, EXP_ROWS)], degs_v)

    def expand(q, carry):
        vec = degs_v[pl.ds(q * 16, 16)] + 1.0
        for k in range(16):
            bc = jnp.full((16,), vec[k], jnp.float32)
            for c in range(D // 16):
                exp_v[q * 16 + k, pl.ds(c * 16, 16)] = bc
        return carry

    lax.fori_loop(0, EXP_ROWS // 16, expand, 0)
    pltpu.sync_copy(exp_v, out_hbm.at[pl.ds(base, EXP_ROWS)])


NSTG = 5
NCHS = NCH_W // NSTG
RING = 4


@functools.partial(
    pl.kernel,
    out_type=jax.ShapeDtypeStruct((NC, NPAD, D), jnp.float32),
    mesh=_mesh,
    scratch_types=[
        pltpu.VMEM((NCHS, CH), jnp.int32),
        pltpu.VMEM((NCHS, CH), jnp.int32),
        pltpu.VMEM((RING, CH, D), jnp.float32),
        pltpu.VMEM_SHARED((NPAD, D), jnp.float32),
        pltpu.SemaphoreType.DMA((RING,)),
    ],
)
def _segsum_kernel(src_hbm, dst_hbm, y_hbm, out_hbm,
                   si_v, di_v, ring_v, acc_sh, semg):
    cid = lax.axis_index("c")
    sid = lax.axis_index("s")
    wid = sid * NC + cid
    _fill_zero_rows(ring_v.at[0], CH)
    for k in range(TILE_ROWS // CH):
        pltpu.sync_copy(ring_v.at[0],
                        acc_sh.at[pl.ds(sid * TILE_ROWS + k * CH, CH)])
    plsc.subcore_barrier()

    for stg in range(NSTG):
        pltpu.sync_copy(src_hbm.at[wid, stg], si_v)
        pltpu.sync_copy(dst_hbm.at[wid, stg], di_v)
        for k in range(RING - 1):
            pltpu.async_copy(y_hbm.at[si_v.at[k]], ring_v.at[k], semg.at[k])

        def itr(g, carry):
            for k in range(RING):
                j = RING * g + k
                bn = (k + RING - 1) % RING

                @pl.when(j + RING - 1 < NCHS)
                def _():
                    pltpu.async_copy(y_hbm.at[si_v.at[j + RING - 1]],
                                     ring_v.at[bn], semg.at[bn])

                pltpu.make_async_copy(y_hbm.at[si_v.at[j]],
                                      ring_v.at[k], semg.at[k]).wait()
                pltpu.sync_copy(ring_v.at[k], acc_sh.at[di_v.at[j]], add=True)
            return carry

        lax.fori_loop(0, NCHS // RING, itr, 0)
        pltpu.make_async_copy(y_hbm.at[si_v.at[NCHS - 1]],
                              ring_v.at[0], semg.at[0]).wait()
        pltpu.sync_copy(ring_v.at[0], acc_sh.at[di_v.at[NCHS - 1]], add=True)

    plsc.subcore_barrier()
    pltpu.sync_copy(acc_sh.at[pl.ds(sid * TILE_ROWS, TILE_ROWS)],
                    out_hbm.at[cid, pl.ds(sid * TILE_ROWS, TILE_ROWS)])


RB = 1280
TC_GRID = NPAD // RB


def _tc_scale_mm(x_ref, w_ref, deg_ref, y_ref):
    dinv = lax.rsqrt(deg_ref[...])
    y_ref[...] = jnp.dot(x_ref[...], w_ref[...],
                         preferred_element_type=jnp.float32) * dinv


_scale_mm = pl.pallas_call(
    _tc_scale_mm,
    grid=(TC_GRID,),
    in_specs=[pl.BlockSpec((RB, D), lambda i: (i, 0)),
              pl.BlockSpec((D, D), lambda i: (0, 0)),
              pl.BlockSpec((RB, D), lambda i: (i, 0))],
    out_specs=pl.BlockSpec((RB, D), lambda i: (i, 0)),
    out_shape=jax.ShapeDtypeStruct((N, D), jnp.float32),
)


def _tc_mid(acc_ref, y_ref, deg_ref, b_ref, w_ref, out_ref):
    dinv = lax.rsqrt(deg_ref[...])
    h = jnp.maximum(
        (acc_ref[0] + acc_ref[1] + y_ref[...]) * dinv + b_ref[...], 0.0)
    out_ref[...] = jnp.dot(h, w_ref[...],
                           preferred_element_type=jnp.float32) * dinv


_mid = pl.pallas_call(
    _tc_mid,
    grid=(TC_GRID,),
    in_specs=[pl.BlockSpec((NC, RB, D), lambda i: (0, i, 0)),
              pl.BlockSpec((RB, D), lambda i: (i, 0)),
              pl.BlockSpec((RB, D), lambda i: (i, 0)),
              pl.BlockSpec((1, D), lambda i: (0, 0)),
              pl.BlockSpec((D, D), lambda i: (0, 0))],
    out_specs=pl.BlockSpec((RB, D), lambda i: (i, 0)),
    out_shape=jax.ShapeDtypeStruct((N, D), jnp.float32),
)


def _tc_final(acc_ref, y_ref, deg_ref, b_ref, out_ref):
    dinv = lax.rsqrt(deg_ref[...])
    out_ref[...] = jnp.maximum(
        (acc_ref[0] + acc_ref[1] + y_ref[...]) * dinv + b_ref[...], 0.0)


_final = pl.pallas_call(
    _tc_final,
    grid=(TC_GRID,),
    in_specs=[pl.BlockSpec((NC, RB, D), lambda i: (0, i, 0)),
              pl.BlockSpec((RB, D), lambda i: (i, 0)),
              pl.BlockSpec((RB, D), lambda i: (i, 0)),
              pl.BlockSpec((1, D), lambda i: (0, 0))],
    out_specs=pl.BlockSpec((RB, D), lambda i: (i, 0)),
    out_shape=jax.ShapeDtypeStruct((N, D), jnp.float32),
)


def kernel(x, edge_index, W1, b1, W2, b2):
    src4d = edge_index[0].reshape(NC * NS, NSTG, NCHS, CH)
    dst4d = edge_index[1].reshape(NC * NS, NSTG, NCHS, CH)
    dst3d_t = edge_index[1].reshape(NS, NCH_T, DCH)
    deg = _deg_kernel(dst3d_t)
    y1 = _scale_mm(x, W1, deg)
    acc1 = _segsum_kernel(src4d, dst4d, y1)
    y2 = _mid(acc1, y1, deg, b1.reshape(1, D), W2)
    acc2 = _segsum_kernel(src4d, dst4d, y2)
    return _final(acc2, y2, deg, b2.reshape(1, D))


# speedup vs baseline: 1.1610x; 1.0468x over previous
import functools

import jax
import jax.numpy as jnp
from jax import lax
from jax.experimental import pallas as pl
from jax.experimental.pallas import tpu as pltpu
from jax.experimental.pallas import tpu_sc as plsc

N = 10000
E = 320000
D = 128
NPAD = 10240
NC = 2
NS = 16
CH = 80
DCH = 80
NCH_W = E // (NC * NS * CH)
NCH_T = E // (NS * DCH)
TILE_ROWS = NPAD // NS
HALF = NPAD // NC
EXP_ROWS = NPAD // (NC * NS)
DEG_GRP = 10

_mesh = plsc.VectorSubcoreMesh(core_axis_name="c", subcore_axis_name="s")


def _fill_zero_rows(ref, nrows):
    z = jnp.zeros((16,), jnp.float32)

    def body(r, carry):
        for c in range(D // 16):
            ref[r, pl.ds(c * 16, 16)] = z
        return carry

    lax.fori_loop(0, nrows, body, 0)


@functools.partial(
    pl.kernel,
    out_type=jax.ShapeDtypeStruct((NPAD, D), jnp.float32),
    mesh=_mesh,
    scratch_types=[
        pltpu.VMEM((NCH_T, DCH), jnp.int32),
        pltpu.VMEM((DCH,), jnp.float32),
        pltpu.VMEM((TILE_ROWS,), jnp.float32),
        pltpu.VMEM((EXP_ROWS,), jnp.float32),
        pltpu.VMEM((EXP_ROWS, D), jnp.float32),
        pltpu.VMEM_SHARED((NPAD,), jnp.float32),
        pltpu.SemaphoreType.DMA,
    ],
)
def _deg_kernel(dst_hbm, out_hbm, di_v, ones_v, zb_v, degs_v, exp_v, deg_sh, sem):
    cid = lax.axis_index("c")
    sid = lax.axis_index("s")
    pltpu.sync_copy(dst_hbm.at[sid], di_v)
    one = jnp.full((16,), 1.0, jnp.float32)
    z = jnp.zeros((16,), jnp.float32)
    for k in range(DCH // 16):
        ones_v[pl.ds(k * 16, 16)] = one
    for k in range(TILE_ROWS // 16):
        zb_v[pl.ds(k * 16, 16)] = z
    pltpu.sync_copy(zb_v, deg_sh.at[pl.ds(sid * TILE_ROWS, TILE_ROWS)])
    plsc.subcore_barrier()

    def grp(g, carry):
        for k in range(DEG_GRP):
            pltpu.async_copy(
                ones_v, deg_sh.at[di_v.at[g * DEG_GRP + k]], sem, add=True)
        for k in range(DEG_GRP):
            pltpu.make_async_copy(
                ones_v, deg_sh.at[di_v.at[g * DEG_GRP + k]], sem).wait()
        return carry

    lax.fori_loop(0, NCH_T // DEG_GRP, grp, 0)
    plsc.subcore_barrier()
    base = cid * HALF + sid * EXP_ROWS
    pltpu.sync_copy(deg_sh.at[pl.ds(base, EXP_ROWS)], degs_v)

    def expand(q, carry):
        vec = degs_v[pl.ds(q * 16, 16)] + 1.0
        for k in range(16):
            bc = jnp.full((16,), vec[k], jnp.float32)
            for c in range(D // 16):
                exp_v[q * 16 + k, pl.ds(c * 16, 16)] = bc
        return carry

    lax.fori_loop(0, EXP_ROWS // 16, expand, 0)
    pltpu.sync_copy(exp_v, out_hbm.at[pl.ds(base, EXP_ROWS)])


NSTG = 5
NCHS = NCH_W // NSTG
RING = 3


@functools.partial(
    pl.kernel,
    out_type=jax.ShapeDtypeStruct((NC, NPAD, D), jnp.float32),
    mesh=_mesh,
    scratch_types=[
        pltpu.VMEM((2, NCHS, CH), jnp.int32),
        pltpu.VMEM((2, NCHS, CH), jnp.int32),
        pltpu.VMEM((RING, CH, D), jnp.float32),
        pltpu.VMEM_SHARED((NPAD, D), jnp.float32),
        pltpu.SemaphoreType.DMA((RING,)),
        pltpu.SemaphoreType.DMA,
    ],
)
def _segsum_kernel(src_hbm, dst_hbm, y_hbm, out_hbm,
                   si_v, di_v, ring_v, acc_sh, semg, semi):
    cid = lax.axis_index("c")
    sid = lax.axis_index("s")
    wid = sid * NC + cid
    _fill_zero_rows(ring_v.at[0], CH)
    for k in range(TILE_ROWS // CH):
        pltpu.sync_copy(ring_v.at[0],
                        acc_sh.at[pl.ds(sid * TILE_ROWS + k * CH, CH)])
    plsc.subcore_barrier()

    pltpu.sync_copy(src_hbm.at[wid, 0], si_v.at[0])
    pltpu.sync_copy(dst_hbm.at[wid, 0], di_v.at[0])
    for k in range(RING - 1):
        pltpu.async_copy(y_hbm.at[si_v.at[0].at[k]], ring_v.at[k], semg.at[k])

    for stg in range(NSTG):
        cur, nxt = stg % 2, (stg + 1) % 2
        scur, dcur, snxt = si_v.at[cur], di_v.at[cur], si_v.at[nxt]
        if stg + 1 < NSTG:
            pltpu.async_copy(src_hbm.at[wid, stg + 1], si_v.at[nxt], semi)
            pltpu.async_copy(dst_hbm.at[wid, stg + 1], di_v.at[nxt], semi)

        def itr(g, carry):
            for k in range(RING):
                j = RING * g + k
                sl = (stg + k) % RING
                pn = (stg + k + RING - 1) % RING

                @pl.when(j + RING - 1 < NCHS)
                def _():
                    pltpu.async_copy(y_hbm.at[scur.at[j + RING - 1]],
                                     ring_v.at[pn], semg.at[pn])

                pltpu.make_async_copy(y_hbm.at[scur.at[j]],
                                      ring_v.at[sl], semg.at[sl]).wait()
                pltpu.sync_copy(ring_v.at[sl], acc_sh.at[dcur.at[j]], add=True)
            return carry

        lax.fori_loop(0, (NCHS - 1) // RING, itr, 0)
        tsl = (stg + NCHS - 1) % RING
        if stg + 1 < NSTG:
            pltpu.make_async_copy(src_hbm.at[wid, stg + 1],
                                  si_v.at[nxt], semi).wait()
            pltpu.make_async_copy(dst_hbm.at[wid, stg + 1],
                                  di_v.at[nxt], semi).wait()
            pltpu.async_copy(y_hbm.at[snxt.at[0]],
                             ring_v.at[(stg + 1) % RING],
                             semg.at[(stg + 1) % RING])
        pltpu.make_async_copy(y_hbm.at[scur.at[NCHS - 1]],
                              ring_v.at[tsl], semg.at[tsl]).wait()
        pltpu.sync_copy(ring_v.at[tsl], acc_sh.at[dcur.at[NCHS - 1]], add=True)
        if stg + 1 < NSTG:
            pltpu.async_copy(y_hbm.at[snxt.at[1]],
                             ring_v.at[(stg + 2) % RING],
                             semg.at[(stg + 2) % RING])

    plsc.subcore_barrier()
    pltpu.sync_copy(acc_sh.at[pl.ds(sid * TILE_ROWS, TILE_ROWS)],
                    out_hbm.at[cid, pl.ds(sid * TILE_ROWS, TILE_ROWS)])


RB = 1280
TC_GRID = NPAD // RB


def _tc_scale_mm(x_ref, w_ref, deg_ref, y_ref):
    dinv = lax.rsqrt(deg_ref[...])
    y_ref[...] = jnp.dot(x_ref[...], w_ref[...],
                         preferred_element_type=jnp.float32) * dinv


_scale_mm = pl.pallas_call(
    _tc_scale_mm,
    grid=(TC_GRID,),
    in_specs=[pl.BlockSpec((RB, D), lambda i: (i, 0)),
              pl.BlockSpec((D, D), lambda i: (0, 0)),
              pl.BlockSpec((RB, D), lambda i: (i, 0))],
    out_specs=pl.BlockSpec((RB, D), lambda i: (i, 0)),
    out_shape=jax.ShapeDtypeStruct((N, D), jnp.float32),
)


def _tc_mid(acc_ref, y_ref, deg_ref, b_ref, w_ref, out_ref):
    dinv = lax.rsqrt(deg_ref[...])
    h = jnp.maximum(
        (acc_ref[0] + acc_ref[1] + y_ref[...]) * dinv + b_ref[...], 0.0)
    out_ref[...] = jnp.dot(h, w_ref[...],
                           preferred_element_type=jnp.float32) * dinv


_mid = pl.pallas_call(
    _tc_mid,
    grid=(TC_GRID,),
    in_specs=[pl.BlockSpec((NC, RB, D), lambda i: (0, i, 0)),
              pl.BlockSpec((RB, D), lambda i: (i, 0)),
              pl.BlockSpec((RB, D), lambda i: (i, 0)),
              pl.BlockSpec((1, D), lambda i: (0, 0)),
              pl.BlockSpec((D, D), lambda i: (0, 0))],
    out_specs=pl.BlockSpec((RB, D), lambda i: (i, 0)),
    out_shape=jax.ShapeDtypeStruct((N, D), jnp.float32),
)


def _tc_final(acc_ref, y_ref, deg_ref, b_ref, out_ref):
    dinv = lax.rsqrt(deg_ref[...])
    out_ref[...] = jnp.maximum(
        (acc_ref[0] + acc_ref[1] + y_ref[...]) * dinv + b_ref[...], 0.0)


_final = pl.pallas_call(
    _tc_final,
    grid=(TC_GRID,),
    in_specs=[pl.BlockSpec((NC, RB, D), lambda i: (0, i, 0)),
              pl.BlockSpec((RB, D), lambda i: (i, 0)),
              pl.BlockSpec((RB, D), lambda i: (i, 0)),
              pl.BlockSpec((1, D), lambda i: (0, 0))],
    out_specs=pl.BlockSpec((RB, D), lambda i: (i, 0)),
    out_shape=jax.ShapeDtypeStruct((N, D), jnp.float32),
)


def kernel(x, edge_index, W1, b1, W2, b2):
    src4d = edge_index[0].reshape(NC * NS, NSTG, NCHS, CH)
    dst4d = edge_index[1].reshape(NC * NS, NSTG, NCHS, CH)
    dst3d_t = edge_index[1].reshape(NS, NCH_T, DCH)
    deg = _deg_kernel(dst3d_t)
    y1 = _scale_mm(x, W1, deg)
    acc1 = _segsum_kernel(src4d, dst4d, y1)
    y2 = _mid(acc1, y1, deg, b1.reshape(1, D), W2)
    acc2 = _segsum_kernel(src4d, dst4d, y2)
    return _final(acc2, y2, deg, b2.reshape(1, D))
